# baseline (speedup 1.0000x reference)
import functools

import jax
import jax.numpy as jnp
from jax import lax
from jax.experimental import pallas as pl
from jax.experimental.pallas import tpu as pltpu
from jax.experimental.pallas import tpu_sc as plsc

N_NODES = 100000
N_PAD = 100096
N_EDGES = 3200000
C = 16
NC = 2
NS = 16
NW = NC * NS
CHUNK = 128
NCH = N_EDGES // CHUNK
NCH_PAD = 25088
NJ = NCH_PAD // NW
NBLK_REAL = 3125
BPW = NCH_PAD // 8 // NW

_mesh = plsc.VectorSubcoreMesh(core_axis_name="c", subcore_axis_name="s")
_params = pltpu.CompilerParams(use_tc_tiling_on_sc=False)

_f32 = jnp.float32
_i32 = jnp.int32


def _zero_rows(ref, nrows):
    def body(i, _):
        ref[i, :] = jnp.zeros((C,), _f32)
        return 0
    lax.fori_loop(0, nrows, body, 0)


def _zero_1d(ref, nvecs):
    def body(i, _):
        ref[pl.ds(i * 16, 16)] = jnp.zeros((16,), _f32)
        return 0
    lax.fori_loop(0, nvecs, body, 0)


@functools.partial(
    pl.kernel,
    out_type=(
        jax.ShapeDtypeStruct((N_PAD,), _f32),
        jax.ShapeDtypeStruct((N_PAD,), _f32),
    ),
    mesh=_mesh,
    compiler_params=_params,
    scratch_types=[
        pltpu.VMEM_SHARED((N_PAD,), _f32),
        pltpu.VMEM((6256,), _f32),
        pltpu.VMEM((8, CHUNK), _i32),
        pltpu.VMEM((8, CHUNK), _f32),
        pltpu.SemaphoreType.DMA,
        pltpu.SemaphoreType.DMA,
    ],
)
def _deg_kernel(row2d, attr2d, degA_hbm, degB_hbm,
                acc, zbuf, rowb8, attrb8, lsem, ssem):
    c = lax.axis_index("c")
    s = lax.axis_index("s")
    wid = c * NS + s
    _zero_1d(zbuf, 6256 // 16)
    pltpu.sync_copy(zbuf, acc.at[pl.ds(s * 6256, 6256)])
    plsc.subcore_barrier()

    cb = wid * 781

    def do_block(ch, nk):
        pltpu.async_copy(row2d.at[pl.ds(ch, nk)], rowb8.at[pl.ds(0, nk)], lsem)
        pltpu.async_copy(attr2d.at[pl.ds(ch, nk)], attrb8.at[pl.ds(0, nk)],
                         lsem)
        pltpu.make_async_copy(row2d.at[pl.ds(0, nk)], rowb8.at[pl.ds(0, nk)],
                              lsem).wait()
        pltpu.make_async_copy(attr2d.at[pl.ds(0, nk)], attrb8.at[pl.ds(0, nk)],
                              lsem).wait()
        for k in range(nk):
            pltpu.async_copy(attrb8.at[k], acc.at[rowb8.at[k]], ssem,
                             add=True)
        for k in range(nk):
            pltpu.make_async_copy(attrb8.at[k], acc.at[rowb8.at[k]],
                                  ssem).wait()

    def blk_body(B, _):
        do_block(cb + 8 * B, 8)
        return 0
    lax.fori_loop(0, 97, blk_body, 0)
    do_block(cb + 776, 5)

    @pl.when(wid < 8)
    def _():
        do_block(24992 + wid, 1)

    plsc.subcore_barrier()
    pltpu.sync_copy(acc.at[pl.ds(s * 6256, 6256)], zbuf)

    @pl.when(c == 0)
    def _():
        pltpu.sync_copy(zbuf, degA_hbm.at[pl.ds(s * 6256, 6256)])

    @pl.when(c == 1)
    def _():
        pltpu.sync_copy(zbuf, degB_hbm.at[pl.ds(s * 6256, 6256)])


@functools.partial(
    pl.kernel,
    out_type=(
        jax.ShapeDtypeStruct((NCH_PAD, 2, CHUNK), _i32),
        jax.ShapeDtypeStruct((NCH_PAD, CHUNK), _f32),
    ),
    mesh=_mesh,
    compiler_params=_params,
    scratch_types=[
        pltpu.VMEM((8, CHUNK), _i32),
        pltpu.VMEM((8, CHUNK), _i32),
        pltpu.VMEM((8, CHUNK), _f32),
        pltpu.VMEM((8, CHUNK), _f32),
        pltpu.VMEM((8, CHUNK), _f32),
        pltpu.VMEM((8, 2, CHUNK), _i32),
        pltpu.VMEM((8, CHUNK), _f32),
        pltpu.SemaphoreType.DMA,
        pltpu.SemaphoreType.DMA,
    ],
)
def _pack_kernel(row2d, col2d, attr2d, degA_hbm, degB_hbm,
                 packi_hbm, packp_hbm,
                 rowb8, colb8, attrb8, da8, db8, pk, pkp, lsem, gsem):
    c = lax.axis_index("c")
    s = lax.axis_index("s")
    wid = c * NS + s

    def blk_body(i, _):
        b = wid * BPW + i
        ch = 8 * b

        @pl.when(b < NBLK_REAL)
        def _():
            pltpu.async_copy(row2d.at[pl.ds(ch, 8)], rowb8, lsem)
            pltpu.async_copy(col2d.at[pl.ds(ch, 8)], colb8, lsem)
            pltpu.async_copy(attr2d.at[pl.ds(ch, 8)], attrb8, lsem)
            pltpu.make_async_copy(row2d.at[pl.ds(0, 8)], rowb8, lsem).wait()
            pltpu.make_async_copy(col2d.at[pl.ds(0, 8)], colb8, lsem).wait()
            pltpu.make_async_copy(attr2d.at[pl.ds(0, 8)], attrb8, lsem).wait()
            for k in range(8):
                pltpu.async_copy(degA_hbm.at[rowb8.at[k]], da8.at[k], gsem)
                pltpu.async_copy(degB_hbm.at[rowb8.at[k]], db8.at[k], gsem)
            for k in range(8):
                pltpu.make_async_copy(degA_hbm.at[rowb8.at[k]], da8.at[k],
                                      gsem).wait()
                pltpu.make_async_copy(degB_hbm.at[rowb8.at[k]], db8.at[k],
                                      gsem).wait()

            def kbody(kk, _):
                for v in range(8):
                    sl = pl.ds(v * 16, 16)
                    pk[kk, 0, sl] = rowb8[kk, sl]
                    pk[kk, 1, sl] = colb8[kk, sl]
                    d = da8[kk, sl] + db8[kk, sl]
                    pkp[kk, sl] = attrb8[kk, sl] / jnp.maximum(d, 1e-12)
                return 0
            lax.fori_loop(0, 8, kbody, 0)
            pltpu.sync_copy(pk, packi_hbm.at[pl.ds(ch, 8)])
            pltpu.sync_copy(pkp, packp_hbm.at[pl.ds(ch, 8)])

        @pl.when(b >= NBLK_REAL)
        def _():
            def zbody(kk, _):
                for r in range(2):
                    for v in range(8):
                        pk[kk, r, pl.ds(v * 16, 16)] = jnp.zeros((16,), _i32)
                for v in range(8):
                    pkp[kk, pl.ds(v * 16, 16)] = jnp.zeros((16,), _f32)
                return 0
            lax.fori_loop(0, 8, zbody, 0)
            pltpu.sync_copy(pk, packi_hbm.at[pl.ds(ch, 8)])
            pltpu.sync_copy(pkp, packp_hbm.at[pl.ds(ch, 8)])
        return 0
    lax.fori_loop(0, BPW, blk_body, 0)


NSLOT = 4


@functools.partial(
    pl.kernel,
    out_type=jax.ShapeDtypeStruct((NC, N_PAD, C), _f32),
    mesh=_mesh,
    compiler_params=_params,
    scratch_types=(
        [pltpu.VMEM_SHARED((N_PAD, C), _f32), pltpu.VMEM((368, C), _f32)]
        + [pltpu.VMEM((2, CHUNK), _i32) for _ in range(NSLOT)]
        + [pltpu.VMEM((CHUNK,), _f32) for _ in range(NSLOT)]
        + [pltpu.VMEM((CHUNK, C), _f32) for _ in range(NSLOT)]
        + [pltpu.SemaphoreType.DMA for _ in range(NSLOT)]
    ),
)
def _step_kernel(packi_hbm, packp_hbm, x_hbm, part_hbm, acc, zbuf, *bufs):
    c = lax.axis_index("c")
    s = lax.axis_index("s")
    wid = c * NS + s
    ebs = bufs[0:NSLOT]
    pbs = bufs[NSLOT:2 * NSLOT]
    rws = bufs[2 * NSLOT:3 * NSLOT]
    sms = bufs[3 * NSLOT:4 * NSLOT]

    _zero_rows(zbuf, 368)

    def zcopy(j, _):
        pltpu.sync_copy(zbuf, acc.at[pl.ds(s * 6256 + j * 368, 368), :])
        return 0
    lax.fori_loop(0, 17, zcopy, 0)
    plsc.subcore_barrier()

    def issue_load(j, b):
        pltpu.async_copy(packi_hbm.at[wid + NW * j], ebs[b], sms[b])
        pltpu.async_copy(packp_hbm.at[wid + NW * j], pbs[b], sms[b])

    def wait_load(b):
        pltpu.make_async_copy(packi_hbm.at[0], ebs[b], sms[b]).wait()
        pltpu.make_async_copy(packp_hbm.at[0], pbs[b], sms[b]).wait()

    def issue_gather(b):
        pltpu.async_copy(x_hbm.at[ebs[b].at[0]], rws[b], sms[b])

    def wait_gather(b):
        pltpu.make_async_copy(x_hbm.at[ebs[b].at[0]], rws[b], sms[b]).wait()

    def issue_scatter(b):
        pltpu.async_copy(rws[b], acc.at[ebs[b].at[1]], sms[b], add=True)

    def wait_scatter(b):
        pltpu.make_async_copy(rws[b], acc.at[ebs[b].at[1]], sms[b]).wait()

    def cgroup(b, g):
        pb, rw = pbs[b], rws[b]
        pv16 = pb[pl.ds(g * 16, 16)]
        for e in range(16):
            idx = jnp.full((16,), e, _i32)
            pv = pv16.at[idx].get(mode="promise_in_bounds")
            r = g * 16 + e
            rw[r, :] = rw[r, :] * pv

    def compute(b):
        def gbody(g, _):
            pb, rw = pbs[b], rws[b]
            pv16 = pb[pl.ds(g * 16, 16)]
            for e in range(16):
                idx = jnp.full((16,), e, _i32)
                pv = pv16.at[idx].get(mode="promise_in_bounds")
                rw[g * 16 + e, :] = rw[g * 16 + e, :] * pv
            return 0
        lax.fori_loop(0, 8, gbody, 0)

    issue_load(0, 0)
    issue_load(1, 1)
    wait_load(0)
    issue_gather(0)
    for j in range(2):
        issue_load(j + 2, j + 2)
        wait_load(j + 1)
        issue_gather(j + 1)
        wait_gather(j)
        compute(j)
        issue_scatter(j)

    def steady(J, _):
        for u in range(4):
            j = 2 + J * 4 + u
            b0 = (2 + u) % 4
            b1 = (3 + u) % 4
            b2 = u
            wait_gather(b0)
            cgroup(b0, 0)
            cgroup(b0, 1)
            wait_scatter(b2)
            cgroup(b0, 2)
            cgroup(b0, 3)
            issue_load(j + 2, b2)
            cgroup(b0, 4)
            cgroup(b0, 5)
            wait_load(b1)
            cgroup(b0, 6)
            issue_gather(b1)
            cgroup(b0, 7)
            issue_scatter(b0)
        return 0
    lax.fori_loop(0, (NJ - 4) // 4, steady, 0)

    wait_scatter(0)
    wait_load(3)
    issue_gather(3)
    wait_gather(2)
    compute(2)
    issue_scatter(2)
    wait_gather(3)
    compute(3)
    issue_scatter(3)
    for b in range(1, 4):
        wait_scatter(b)

    plsc.subcore_barrier()

    def wcopy(j, _):
        base = s * 6256 + j * 368
        pltpu.sync_copy(acc.at[pl.ds(base, 368), :], zbuf)
        pltpu.sync_copy(zbuf, part_hbm.at[c, pl.ds(base, 368), :])
        return 0
    lax.fori_loop(0, 17, wcopy, 0)


ROWS_PER_W = N_PAD // NW
CB_CHUNK = 1564
CB_N = ROWS_PER_W // CB_CHUNK


@functools.partial(
    pl.kernel,
    out_type=(
        jax.ShapeDtypeStruct((N_PAD, C), _f32),
        jax.ShapeDtypeStruct((N_PAD, C), _f32),
    ),
    mesh=_mesh,
    compiler_params=_params,
    scratch_types=[
        pltpu.VMEM((16,), _f32),
        pltpu.VMEM((CB_CHUNK, C), _f32),
        pltpu.VMEM((CB_CHUNK, C), _f32),
        pltpu.VMEM((CB_CHUNK, C), _f32),
        pltpu.SemaphoreType.DMA,
    ],
)
def _combine_kernel(part_hbm, out_old_hbm, wk_hbm, x_hbm, out_hbm,
                    wkb, ab, bb, ob, lsem):
    c = lax.axis_index("c")
    s = lax.axis_index("s")
    wid = c * NS + s
    pltpu.sync_copy(wk_hbm, wkb)

    def chunk_body(ci, _):
        base = wid * ROWS_PER_W + ci * CB_CHUNK
        pltpu.async_copy(part_hbm.at[0, pl.ds(base, CB_CHUNK), :], ab, lsem)
        pltpu.async_copy(part_hbm.at[1, pl.ds(base, CB_CHUNK), :], bb, lsem)
        pltpu.async_copy(out_old_hbm.at[pl.ds(base, CB_CHUNK), :], ob, lsem)
        pltpu.make_async_copy(part_hbm.at[0, pl.ds(0, CB_CHUNK), :], ab,
                              lsem).wait()
        pltpu.make_async_copy(part_hbm.at[0, pl.ds(0, CB_CHUNK), :], bb,
                              lsem).wait()
        pltpu.make_async_copy(out_old_hbm.at[pl.ds(0, CB_CHUNK), :], ob,
                              lsem).wait()
        wk = wkb[...]

        def rbody(i, _):
            xv = ab[i, :] + bb[i, :]
            ab[i, :] = xv
            ob[i, :] = ob[i, :] + xv * wk
            return 0
        lax.fori_loop(0, CB_CHUNK, rbody, 0)
        pltpu.sync_copy(ab, x_hbm.at[pl.ds(base, CB_CHUNK), :])
        pltpu.sync_copy(ob, out_hbm.at[pl.ds(base, CB_CHUNK), :])
        return 0
    lax.fori_loop(0, CB_N, chunk_body, 0)


def kernel(edge_index, edge_attr, target, weight):
    row2d = edge_index[0].astype(_i32).reshape(NCH, CHUNK)
    col2d = edge_index[1].astype(_i32).reshape(NCH, CHUNK)
    attr2d = edge_attr.astype(_f32).reshape(NCH, CHUNK)
    degA, degB = _deg_kernel(row2d, attr2d)
    packi, packp = _pack_kernel(row2d, col2d, attr2d, degA, degB)
    x = jnp.pad(target, ((0, N_PAD - N_NODES), (0, 0)))
    out = jnp.zeros((N_PAD, C), _f32)
    for k in range(weight.shape[1]):
        parts = _step_kernel(packi, packp, x)
        x, out = _combine_kernel(parts, out, weight[:, k])
    return out[:N_NODES]


# speedup vs baseline: 1.9835x; 1.9835x over previous
import functools

import jax
import jax.numpy as jnp
from jax import lax
from jax.experimental import pallas as pl
from jax.experimental.pallas import tpu as pltpu
from jax.experimental.pallas import tpu_sc as plsc

N_NODES = 100000
N_PAD = 100096
N_EDGES = 3200000
C = 16
NC = 2
NS = 16
NW = NC * NS
CHUNK = 128
NCH = N_EDGES // CHUNK
NCH_PAD = 25088
NJ = NCH_PAD // NW
NBLK_REAL = 3125
BPW = NCH_PAD // 8 // NW

_mesh = plsc.VectorSubcoreMesh(core_axis_name="c", subcore_axis_name="s")
_params = pltpu.CompilerParams(use_tc_tiling_on_sc=False)

_f32 = jnp.float32
_i32 = jnp.int32


def _zero_rows(ref, nrows):
    def body(i, _):
        ref[i, :] = jnp.zeros((C,), _f32)
        return 0
    lax.fori_loop(0, nrows, body, 0)


def _zero_1d(ref, nvecs):
    def body(i, _):
        ref[pl.ds(i * 16, 16)] = jnp.zeros((16,), _f32)
        return 0
    lax.fori_loop(0, nvecs, body, 0)


@functools.partial(
    pl.kernel,
    out_type=(
        jax.ShapeDtypeStruct((N_PAD,), _f32),
        jax.ShapeDtypeStruct((N_PAD,), _f32),
    ),
    mesh=_mesh,
    compiler_params=_params,
    scratch_types=[
        pltpu.VMEM_SHARED((N_PAD,), _f32),
        pltpu.VMEM((6256,), _f32),
        pltpu.VMEM((8, CHUNK), _i32),
        pltpu.VMEM((8, CHUNK), _f32),
        pltpu.SemaphoreType.DMA,
        pltpu.SemaphoreType.DMA,
    ],
)
def _deg_kernel(row2d, attr2d, degA_hbm, degB_hbm,
                acc, zbuf, rowb8, attrb8, lsem, ssem):
    c = lax.axis_index("c")
    s = lax.axis_index("s")
    wid = c * NS + s
    _zero_1d(zbuf, 6256 // 16)
    pltpu.sync_copy(zbuf, acc.at[pl.ds(s * 6256, 6256)])
    plsc.subcore_barrier()

    cb = wid * 781

    def do_block(ch, nk):
        pltpu.async_copy(row2d.at[pl.ds(ch, nk)], rowb8.at[pl.ds(0, nk)], lsem)
        pltpu.async_copy(attr2d.at[pl.ds(ch, nk)], attrb8.at[pl.ds(0, nk)],
                         lsem)
        pltpu.make_async_copy(row2d.at[pl.ds(0, nk)], rowb8.at[pl.ds(0, nk)],
                              lsem).wait()
        pltpu.make_async_copy(attr2d.at[pl.ds(0, nk)], attrb8.at[pl.ds(0, nk)],
                              lsem).wait()
        for k in range(nk):
            pltpu.async_copy(attrb8.at[k], acc.at[rowb8.at[k]], ssem,
                             add=True)
        for k in range(nk):
            pltpu.make_async_copy(attrb8.at[k], acc.at[rowb8.at[k]],
                                  ssem).wait()

    def blk_body(B, _):
        do_block(cb + 8 * B, 8)
        return 0
    lax.fori_loop(0, 97, blk_body, 0)
    do_block(cb + 776, 5)

    @pl.when(wid < 8)
    def _():
        do_block(24992 + wid, 1)

    plsc.subcore_barrier()
    pltpu.sync_copy(acc.at[pl.ds(s * 6256, 6256)], zbuf)

    @pl.when(c == 0)
    def _():
        pltpu.sync_copy(zbuf, degA_hbm.at[pl.ds(s * 6256, 6256)])

    @pl.when(c == 1)
    def _():
        pltpu.sync_copy(zbuf, degB_hbm.at[pl.ds(s * 6256, 6256)])


@functools.partial(
    pl.kernel,
    out_type=(
        jax.ShapeDtypeStruct((NCH_PAD, 2, CHUNK), _i32),
        jax.ShapeDtypeStruct((NCH_PAD, CHUNK), _f32),
    ),
    mesh=_mesh,
    compiler_params=_params,
    scratch_types=[
        pltpu.VMEM((8, CHUNK), _i32),
        pltpu.VMEM((8, CHUNK), _i32),
        pltpu.VMEM((8, CHUNK), _f32),
        pltpu.VMEM((8, CHUNK), _f32),
        pltpu.VMEM((8, CHUNK), _f32),
        pltpu.VMEM((8, 2, CHUNK), _i32),
        pltpu.VMEM((8, CHUNK), _f32),
        pltpu.SemaphoreType.DMA,
        pltpu.SemaphoreType.DMA,
    ],
)
def _pack_kernel(row2d, col2d, attr2d, degA_hbm, degB_hbm,
                 packi_hbm, packp_hbm,
                 rowb8, colb8, attrb8, da8, db8, pk, pkp, lsem, gsem):
    c = lax.axis_index("c")
    s = lax.axis_index("s")
    wid = c * NS + s

    def blk_body(i, _):
        b = wid * BPW + i
        ch = 8 * b

        @pl.when(b < NBLK_REAL)
        def _():
            pltpu.async_copy(row2d.at[pl.ds(ch, 8)], rowb8, lsem)
            pltpu.async_copy(col2d.at[pl.ds(ch, 8)], colb8, lsem)
            pltpu.async_copy(attr2d.at[pl.ds(ch, 8)], attrb8, lsem)
            pltpu.make_async_copy(row2d.at[pl.ds(0, 8)], rowb8, lsem).wait()
            pltpu.make_async_copy(col2d.at[pl.ds(0, 8)], colb8, lsem).wait()
            pltpu.make_async_copy(attr2d.at[pl.ds(0, 8)], attrb8, lsem).wait()
            for k in range(8):
                pltpu.async_copy(degA_hbm.at[rowb8.at[k]], da8.at[k], gsem)
                pltpu.async_copy(degB_hbm.at[rowb8.at[k]], db8.at[k], gsem)
            for k in range(8):
                pltpu.make_async_copy(degA_hbm.at[rowb8.at[k]], da8.at[k],
                                      gsem).wait()
                pltpu.make_async_copy(degB_hbm.at[rowb8.at[k]], db8.at[k],
                                      gsem).wait()

            def kbody(kk, _):
                for v in range(8):
                    sl = pl.ds(v * 16, 16)
                    pk[kk, 0, sl] = rowb8[kk, sl]
                    pk[kk, 1, sl] = colb8[kk, sl]
                    d = da8[kk, sl] + db8[kk, sl]
                    pkp[kk, sl] = attrb8[kk, sl] / jnp.maximum(d, 1e-12)
                return 0
            lax.fori_loop(0, 8, kbody, 0)
            pltpu.sync_copy(pk, packi_hbm.at[pl.ds(ch, 8)])
            pltpu.sync_copy(pkp, packp_hbm.at[pl.ds(ch, 8)])

        @pl.when(b >= NBLK_REAL)
        def _():
            def zbody(kk, _):
                for r in range(2):
                    for v in range(8):
                        pk[kk, r, pl.ds(v * 16, 16)] = jnp.zeros((16,), _i32)
                for v in range(8):
                    pkp[kk, pl.ds(v * 16, 16)] = jnp.zeros((16,), _f32)
                return 0
            lax.fori_loop(0, 8, zbody, 0)
            pltpu.sync_copy(pk, packi_hbm.at[pl.ds(ch, 8)])
            pltpu.sync_copy(pkp, packp_hbm.at[pl.ds(ch, 8)])
        return 0
    lax.fori_loop(0, BPW, blk_body, 0)


NSLOT = 4
SUP = 2
NT = NJ // SUP


@functools.partial(
    pl.kernel,
    out_type=jax.ShapeDtypeStruct((NC, N_PAD, C), _f32),
    mesh=_mesh,
    compiler_params=_params,
    scratch_types=(
        [pltpu.VMEM_SHARED((N_PAD, C), _f32), pltpu.VMEM((368, C), _f32)]
        + [pltpu.VMEM((SUP, 2, CHUNK), _i32) for _ in range(NSLOT)]
        + [pltpu.VMEM((SUP, CHUNK), _f32) for _ in range(NSLOT)]
        + [pltpu.VMEM((SUP * CHUNK, C), _f32) for _ in range(NSLOT)]
        + [pltpu.SemaphoreType.DMA for _ in range(NSLOT)]
    ),
)
def _step_kernel(packi_hbm, packp_hbm, x_hbm, part_hbm, acc, zbuf, *bufs):
    c = lax.axis_index("c")
    s = lax.axis_index("s")
    wid = c * NS + s
    ebs = bufs[0:NSLOT]
    pbs = bufs[NSLOT:2 * NSLOT]
    rws = bufs[2 * NSLOT:3 * NSLOT]
    sms = bufs[3 * NSLOT:4 * NSLOT]
    cb = wid * NJ

    _zero_rows(zbuf, 368)

    def zcopy(j, _):
        pltpu.sync_copy(zbuf, acc.at[pl.ds(s * 6256 + j * 368, 368), :])
        return 0
    lax.fori_loop(0, 17, zcopy, 0)
    plsc.subcore_barrier()

    def issue_load(t, b):
        base = cb + SUP * t
        pltpu.async_copy(packi_hbm.at[pl.ds(base, SUP)], ebs[b], sms[b])
        pltpu.async_copy(packp_hbm.at[pl.ds(base, SUP)], pbs[b], sms[b])

    def wait_load(b):
        pltpu.make_async_copy(packi_hbm.at[pl.ds(0, SUP)], ebs[b],
                              sms[b]).wait()
        pltpu.make_async_copy(packp_hbm.at[pl.ds(0, SUP)], pbs[b],
                              sms[b]).wait()

    def issue_gather(b):
        for k in range(SUP):
            pltpu.async_copy(x_hbm.at[ebs[b].at[k, 0]],
                             rws[b].at[pl.ds(k * CHUNK, CHUNK), :], sms[b])

    def wait_gather(b):
        for k in range(SUP):
            pltpu.make_async_copy(x_hbm.at[ebs[b].at[k, 0]],
                                  rws[b].at[pl.ds(k * CHUNK, CHUNK), :],
                                  sms[b]).wait()

    def issue_scatter(b):
        for k in range(SUP):
            pltpu.async_copy(rws[b].at[pl.ds(k * CHUNK, CHUNK), :],
                             acc.at[ebs[b].at[k, 1]], sms[b], add=True)

    def wait_scatter(b):
        for k in range(SUP):
            pltpu.make_async_copy(rws[b].at[pl.ds(k * CHUNK, CHUNK), :],
                                  acc.at[ebs[b].at[k, 1]], sms[b]).wait()

    def compute(b):
        pb, rw = pbs[b], rws[b]

        def kbody(k, _):
            def gbody(g, _):
                pv16 = pb[k, pl.ds(g * 16, 16)]
                base = k * CHUNK + g * 16
                for e in range(16):
                    idx = jnp.full((16,), e, _i32)
                    pv = pv16.at[idx].get(mode="promise_in_bounds")
                    rw[base + e, :] = rw[base + e, :] * pv
                return 0
            lax.fori_loop(0, 8, gbody, 0)
            return 0
        lax.fori_loop(0, SUP, kbody, 0)

    issue_load(0, 0)
    issue_load(1, 1)
    wait_load(0)
    issue_gather(0)
    for t in range(2):
        issue_load(t + 2, t + 2)
        wait_load(t + 1)
        issue_gather(t + 1)
        wait_gather(t)
        compute(t)
        issue_scatter(t)

    def steady(J, _):
        for u in range(4):
            t = 2 + J * 4 + u
            b0 = (2 + u) % 4
            b1 = (3 + u) % 4
            b2 = u
            wait_scatter(b2)
            issue_load(t + 2, b2)
            wait_load(b1)
            issue_gather(b1)
            wait_gather(b0)
            compute(b0)
            issue_scatter(b0)
        return 0
    lax.fori_loop(0, (NT - 4) // 4, steady, 0)

    wait_scatter(0)
    wait_load(3)
    issue_gather(3)
    wait_gather(2)
    compute(2)
    issue_scatter(2)
    wait_gather(3)
    compute(3)
    issue_scatter(3)
    for b in range(1, 4):
        wait_scatter(b)

    plsc.subcore_barrier()

    def wcopy(j, _):
        base = s * 6256 + j * 368
        pltpu.sync_copy(acc.at[pl.ds(base, 368), :], zbuf)
        pltpu.sync_copy(zbuf, part_hbm.at[c, pl.ds(base, 368), :])
        return 0
    lax.fori_loop(0, 17, wcopy, 0)


ROWS_PER_W = N_PAD // NW
CB_CHUNK = 1564
CB_N = ROWS_PER_W // CB_CHUNK


@functools.partial(
    pl.kernel,
    out_type=(
        jax.ShapeDtypeStruct((N_PAD, C), _f32),
        jax.ShapeDtypeStruct((N_PAD, C), _f32),
    ),
    mesh=_mesh,
    compiler_params=_params,
    scratch_types=[
        pltpu.VMEM((16,), _f32),
        pltpu.VMEM((CB_CHUNK, C), _f32),
        pltpu.VMEM((CB_CHUNK, C), _f32),
        pltpu.VMEM((CB_CHUNK, C), _f32),
        pltpu.SemaphoreType.DMA,
    ],
)
def _combine_kernel(part_hbm, out_old_hbm, wk_hbm, x_hbm, out_hbm,
                    wkb, ab, bb, ob, lsem):
    c = lax.axis_index("c")
    s = lax.axis_index("s")
    wid = c * NS + s
    pltpu.sync_copy(wk_hbm, wkb)

    def chunk_body(ci, _):
        base = wid * ROWS_PER_W + ci * CB_CHUNK
        pltpu.async_copy(part_hbm.at[0, pl.ds(base, CB_CHUNK), :], ab, lsem)
        pltpu.async_copy(part_hbm.at[1, pl.ds(base, CB_CHUNK), :], bb, lsem)
        pltpu.async_copy(out_old_hbm.at[pl.ds(base, CB_CHUNK), :], ob, lsem)
        pltpu.make_async_copy(part_hbm.at[0, pl.ds(0, CB_CHUNK), :], ab,
                              lsem).wait()
        pltpu.make_async_copy(part_hbm.at[0, pl.ds(0, CB_CHUNK), :], bb,
                              lsem).wait()
        pltpu.make_async_copy(out_old_hbm.at[pl.ds(0, CB_CHUNK), :], ob,
                              lsem).wait()
        wk = wkb[...]

        def rbody(i, _):
            xv = ab[i, :] + bb[i, :]
            ab[i, :] = xv
            ob[i, :] = ob[i, :] + xv * wk
            return 0
        lax.fori_loop(0, CB_CHUNK, rbody, 0)
        pltpu.sync_copy(ab, x_hbm.at[pl.ds(base, CB_CHUNK), :])
        pltpu.sync_copy(ob, out_hbm.at[pl.ds(base, CB_CHUNK), :])
        return 0
    lax.fori_loop(0, CB_N, chunk_body, 0)


def kernel(edge_index, edge_attr, target, weight):
    row2d = edge_index[0].astype(_i32).reshape(NCH, CHUNK)
    col2d = edge_index[1].astype(_i32).reshape(NCH, CHUNK)
    attr2d = edge_attr.astype(_f32).reshape(NCH, CHUNK)
    degA, degB = _deg_kernel(row2d, attr2d)
    packi, packp = _pack_kernel(row2d, col2d, attr2d, degA, degB)
    x = jnp.pad(target, ((0, N_PAD - N_NODES), (0, 0)))
    out = jnp.zeros((N_PAD, C), _f32)
    for k in range(weight.shape[1]):
        parts = _step_kernel(packi, packp, x)
        x, out = _combine_kernel(parts, out, weight[:, k])
    return out[:N_NODES]


# speedup vs baseline: 2.1002x; 1.0589x over previous
import functools

import jax
import jax.numpy as jnp
from jax import lax
from jax.experimental import pallas as pl
from jax.experimental.pallas import tpu as pltpu
from jax.experimental.pallas import tpu_sc as plsc

N_NODES = 100000
N_PAD = 100096
N_EDGES = 3200000
C = 16
NC = 2
NS = 16
NW = NC * NS
CHUNK = 128
NCH = N_EDGES // CHUNK
NCH_PAD = 25088
NJ = NCH_PAD // NW
NBLK_REAL = 3125
BPW = NCH_PAD // 8 // NW

_mesh = plsc.VectorSubcoreMesh(core_axis_name="c", subcore_axis_name="s")
_params = pltpu.CompilerParams(use_tc_tiling_on_sc=False)

_f32 = jnp.float32
_i32 = jnp.int32


def _zero_rows(ref, nrows):
    def body(i, _):
        ref[i, :] = jnp.zeros((C,), _f32)
        return 0
    lax.fori_loop(0, nrows, body, 0)


def _zero_1d(ref, nvecs):
    def body(i, _):
        ref[pl.ds(i * 16, 16)] = jnp.zeros((16,), _f32)
        return 0
    lax.fori_loop(0, nvecs, body, 0)


@functools.partial(
    pl.kernel,
    out_type=(
        jax.ShapeDtypeStruct((N_PAD,), _f32),
        jax.ShapeDtypeStruct((N_PAD,), _f32),
    ),
    mesh=_mesh,
    compiler_params=_params,
    scratch_types=[
        pltpu.VMEM_SHARED((N_PAD,), _f32),
        pltpu.VMEM((6256,), _f32),
        pltpu.VMEM((8, CHUNK), _i32),
        pltpu.VMEM((8, CHUNK), _i32),
        pltpu.VMEM((8, CHUNK), _f32),
        pltpu.VMEM((8, CHUNK), _f32),
        pltpu.SemaphoreType.DMA,
        pltpu.SemaphoreType.DMA,
    ],
)
def _deg_kernel(row2d, attr2d, degA_hbm, degB_hbm,
                acc, zbuf, rb0, rb1, ab0, ab1, sm0, sm1):
    c = lax.axis_index("c")
    s = lax.axis_index("s")
    wid = c * NS + s
    rbs, abs_, sms = (rb0, rb1), (ab0, ab1), (sm0, sm1)
    _zero_1d(zbuf, 6256 // 16)
    pltpu.sync_copy(zbuf, acc.at[pl.ds(s * 6256, 6256)])
    plsc.subcore_barrier()

    cb = wid * 781

    def issue_load(B, a):
        pltpu.async_copy(row2d.at[pl.ds(cb + 8 * B, 8)], rbs[a], sms[a])
        pltpu.async_copy(attr2d.at[pl.ds(cb + 8 * B, 8)], abs_[a], sms[a])

    def wait_load(a):
        pltpu.make_async_copy(row2d.at[pl.ds(0, 8)], rbs[a], sms[a]).wait()
        pltpu.make_async_copy(attr2d.at[pl.ds(0, 8)], abs_[a], sms[a]).wait()

    def issue_scat(a, nk=8):
        for k in range(nk):
            pltpu.async_copy(abs_[a].at[k], acc.at[rbs[a].at[k]], sms[a],
                             add=True)

    def wait_scat(a, nk=8):
        for k in range(nk):
            pltpu.make_async_copy(abs_[a].at[k], acc.at[rbs[a].at[k]],
                                  sms[a]).wait()

    issue_load(0, 0)
    issue_load(1, 1)

    def dloop(J, _):
        for u in range(2):
            B = 2 * J + u
            a = u
            wait_load(a)
            issue_scat(a)
            wait_scat(a)
            issue_load(B + 2, a)
        return 0
    lax.fori_loop(0, 47, dloop, 0)
    for a in (0, 1):
        wait_load(a)
        issue_scat(a)
        wait_scat(a)
    issue_load(96, 0)
    wait_load(0)
    issue_scat(0)
    wait_scat(0)
    pltpu.async_copy(row2d.at[pl.ds(cb + 776, 5)], rbs[1].at[pl.ds(0, 5)],
                     sms[1])
    pltpu.async_copy(attr2d.at[pl.ds(cb + 776, 5)], abs_[1].at[pl.ds(0, 5)],
                     sms[1])
    pltpu.make_async_copy(row2d.at[pl.ds(0, 5)], rbs[1].at[pl.ds(0, 5)],
                          sms[1]).wait()
    pltpu.make_async_copy(attr2d.at[pl.ds(0, 5)], abs_[1].at[pl.ds(0, 5)],
                          sms[1]).wait()
    issue_scat(1, 5)
    wait_scat(1, 5)

    @pl.when(wid < 8)
    def _():
        pltpu.async_copy(row2d.at[pl.ds(24992 + wid, 1)],
                         rbs[0].at[pl.ds(0, 1)], sms[0])
        pltpu.async_copy(attr2d.at[pl.ds(24992 + wid, 1)],
                         abs_[0].at[pl.ds(0, 1)], sms[0])
        pltpu.make_async_copy(row2d.at[pl.ds(0, 1)], rbs[0].at[pl.ds(0, 1)],
                              sms[0]).wait()
        pltpu.make_async_copy(attr2d.at[pl.ds(0, 1)], abs_[0].at[pl.ds(0, 1)],
                              sms[0]).wait()
        issue_scat(0, 1)
        wait_scat(0, 1)

    plsc.subcore_barrier()
    pltpu.sync_copy(acc.at[pl.ds(s * 6256, 6256)], zbuf)

    @pl.when(c == 0)
    def _():
        pltpu.sync_copy(zbuf, degA_hbm.at[pl.ds(s * 6256, 6256)])

    @pl.when(c == 1)
    def _():
        pltpu.sync_copy(zbuf, degB_hbm.at[pl.ds(s * 6256, 6256)])


@functools.partial(
    pl.kernel,
    out_type=jax.ShapeDtypeStruct((N_PAD,), _f32),
    mesh=_mesh,
    compiler_params=_params,
    scratch_types=[
        pltpu.VMEM((6256,), _f32),
        pltpu.VMEM((6256,), _f32),
    ],
)
def _degsum_kernel(degA_hbm, degB_hbm, deg_hbm, ab, bb):
    c = lax.axis_index("c")
    s = lax.axis_index("s")

    @pl.when(c == 0)
    def _():
        base = s * 6256
        pltpu.sync_copy(degA_hbm.at[pl.ds(base, 6256)], ab)
        pltpu.sync_copy(degB_hbm.at[pl.ds(base, 6256)], bb)

        def body(i, _):
            sl = pl.ds(i * 16, 16)
            ab[sl] = ab[sl] + bb[sl]
            return 0
        lax.fori_loop(0, 6256 // 16, body, 0)
        pltpu.sync_copy(ab, deg_hbm.at[pl.ds(base, 6256)])


@functools.partial(
    pl.kernel,
    out_type=(
        jax.ShapeDtypeStruct((NCH_PAD, 2, CHUNK), _i32),
        jax.ShapeDtypeStruct((NCH_PAD, CHUNK), _f32),
    ),
    mesh=_mesh,
    compiler_params=_params,
    scratch_types=(
        [pltpu.VMEM((8, CHUNK), _i32) for _ in range(4)]
        + [pltpu.VMEM((8, CHUNK), _f32) for _ in range(4)]
        + [pltpu.VMEM((8, 2, CHUNK), _i32) for _ in range(2)]
        + [pltpu.VMEM((8, CHUNK), _f32) for _ in range(2)]
        + [pltpu.SemaphoreType.DMA for _ in range(2)]
    ),
)
def _pack_kernel(row2d, col2d, attr2d, deg_hbm, packi_hbm, packp_hbm, *bufs):
    c = lax.axis_index("c")
    s = lax.axis_index("s")
    wid = c * NS + s
    rbs = bufs[0:2]
    cbs = bufs[2:4]
    abs_ = bufs[4:6]
    dbs = bufs[6:8]
    pks = bufs[8:10]
    pps = bufs[10:12]
    sms = bufs[12:14]

    def issue_load(b, a):
        ch = jnp.minimum(8 * b, NCH - 8)
        pltpu.async_copy(row2d.at[pl.ds(ch, 8)], rbs[a], sms[a])
        pltpu.async_copy(col2d.at[pl.ds(ch, 8)], cbs[a], sms[a])
        pltpu.async_copy(attr2d.at[pl.ds(ch, 8)], abs_[a], sms[a])

    def wait_load(a):
        pltpu.make_async_copy(row2d.at[pl.ds(0, 8)], rbs[a], sms[a]).wait()
        pltpu.make_async_copy(col2d.at[pl.ds(0, 8)], cbs[a], sms[a]).wait()
        pltpu.make_async_copy(attr2d.at[pl.ds(0, 8)], abs_[a], sms[a]).wait()

    def issue_gath(a):
        for k in range(8):
            pltpu.async_copy(deg_hbm.at[rbs[a].at[k]], dbs[a].at[k], sms[a])

    def wait_gath(a):
        for k in range(8):
            pltpu.make_async_copy(deg_hbm.at[rbs[a].at[k]], dbs[a].at[k],
                                  sms[a]).wait()

    def compute_and_write(b, a):
        pk, pkp = pks[a], pps[a]

        @pl.when(b < NBLK_REAL)
        def _():
            def kbody(kk, _):
                for v in range(8):
                    sl = pl.ds(v * 16, 16)
                    pk[kk, 0, sl] = rbs[a][kk, sl]
                    pk[kk, 1, sl] = cbs[a][kk, sl]
                    pkp[kk, sl] = abs_[a][kk, sl] / jnp.maximum(
                        dbs[a][kk, sl], 1e-12)
                return 0
            lax.fori_loop(0, 8, kbody, 0)

        @pl.when(b >= NBLK_REAL)
        def _():
            def zbody(kk, _):
                for r in range(2):
                    for v in range(8):
                        pk[kk, r, pl.ds(v * 16, 16)] = jnp.zeros((16,), _i32)
                for v in range(8):
                    pkp[kk, pl.ds(v * 16, 16)] = jnp.zeros((16,), _f32)
                return 0
            lax.fori_loop(0, 8, zbody, 0)
        pltpu.async_copy(pk, packi_hbm.at[pl.ds(8 * b, 8)], sms[a])
        pltpu.async_copy(pkp, packp_hbm.at[pl.ds(8 * b, 8)], sms[a])

    def wait_write(a):
        pltpu.make_async_copy(pks[a], packi_hbm.at[pl.ds(0, 8)],
                              sms[a]).wait()
        pltpu.make_async_copy(pps[a], packp_hbm.at[pl.ds(0, 8)],
                              sms[a]).wait()

    b0 = wid * BPW
    issue_load(b0, 0)
    wait_load(0)
    issue_gath(0)
    issue_load(b0 + 1, 1)
    wait_gath(0)
    compute_and_write(b0, 0)

    def ploop(J, _):
        for u in range(2):
            i = 1 + 2 * J + u
            a = (1 + u) % 2
            o = u
            wait_load(a)
            issue_gath(a)
            wait_write(o)
            issue_load(b0 + i + 1, o)
            wait_gath(a)
            compute_and_write(b0 + i, a)
        return 0
    lax.fori_loop(0, 48, ploop, 0)
    wait_load(1)
    issue_gath(1)
    wait_write(0)
    wait_gath(1)
    compute_and_write(b0 + 97, 1)
    wait_write(1)


NSLOT = 4
SUP = 2
NT = NJ // SUP


@functools.partial(
    pl.kernel,
    out_type=jax.ShapeDtypeStruct((NC, N_PAD, C), _f32),
    mesh=_mesh,
    compiler_params=_params,
    scratch_types=(
        [pltpu.VMEM_SHARED((N_PAD, C), _f32), pltpu.VMEM((368, C), _f32)]
        + [pltpu.VMEM((SUP, 2, CHUNK), _i32) for _ in range(NSLOT)]
        + [pltpu.VMEM((SUP, CHUNK), _f32) for _ in range(NSLOT)]
        + [pltpu.VMEM((SUP * CHUNK, C), _f32) for _ in range(NSLOT)]
        + [pltpu.SemaphoreType.DMA for _ in range(NSLOT)]
    ),
)
def _step_kernel(packi_hbm, packp_hbm, x_hbm, part_hbm, acc, zbuf, *bufs):
    c = lax.axis_index("c")
    s = lax.axis_index("s")
    wid = c * NS + s
    ebs = bufs[0:NSLOT]
    pbs = bufs[NSLOT:2 * NSLOT]
    rws = bufs[2 * NSLOT:3 * NSLOT]
    sms = bufs[3 * NSLOT:4 * NSLOT]
    cb = wid * NJ

    _zero_rows(zbuf, 368)

    def zcopy(j, _):
        pltpu.sync_copy(zbuf, acc.at[pl.ds(s * 6256 + j * 368, 368), :])
        return 0
    lax.fori_loop(0, 17, zcopy, 0)
    plsc.subcore_barrier()

    def issue_load(t, b):
        base = cb + SUP * t
        pltpu.async_copy(packi_hbm.at[pl.ds(base, SUP)], ebs[b], sms[b])
        pltpu.async_copy(packp_hbm.at[pl.ds(base, SUP)], pbs[b], sms[b])

    def wait_load(b):
        pltpu.make_async_copy(packi_hbm.at[pl.ds(0, SUP)], ebs[b],
                              sms[b]).wait()
        pltpu.make_async_copy(packp_hbm.at[pl.ds(0, SUP)], pbs[b],
                              sms[b]).wait()

    def issue_gather(b):
        for k in range(SUP):
            pltpu.async_copy(x_hbm.at[ebs[b].at[k, 0]],
                             rws[b].at[pl.ds(k * CHUNK, CHUNK), :], sms[b])

    def wait_gather(b):
        for k in range(SUP):
            pltpu.make_async_copy(x_hbm.at[ebs[b].at[k, 0]],
                                  rws[b].at[pl.ds(k * CHUNK, CHUNK), :],
                                  sms[b]).wait()

    def issue_scatter(b):
        for k in range(SUP):
            pltpu.async_copy(rws[b].at[pl.ds(k * CHUNK, CHUNK), :],
                             acc.at[ebs[b].at[k, 1]], sms[b], add=True)

    def wait_scatter(b):
        for k in range(SUP):
            pltpu.make_async_copy(rws[b].at[pl.ds(k * CHUNK, CHUNK), :],
                                  acc.at[ebs[b].at[k, 1]], sms[b]).wait()

    def compute(b):
        pb, rw = pbs[b], rws[b]

        def kbody(k, _):
            def gbody(g, _):
                pv16 = pb[k, pl.ds(g * 16, 16)]
                base = k * CHUNK + g * 16
                for e in range(16):
                    idx = jnp.full((16,), e, _i32)
                    pv = pv16.at[idx].get(mode="promise_in_bounds")
                    rw[base + e, :] = rw[base + e, :] * pv
                return 0
            lax.fori_loop(0, 8, gbody, 0)
            return 0
        lax.fori_loop(0, SUP, kbody, 0)

    issue_load(0, 0)
    issue_load(1, 1)
    wait_load(0)
    issue_gather(0)
    for t in range(2):
        issue_load(t + 2, t + 2)
        wait_load(t + 1)
        issue_gather(t + 1)
        wait_gather(t)
        compute(t)
        issue_scatter(t)

    def steady(J, _):
        for u in range(4):
            t = 2 + J * 4 + u
            b0 = (2 + u) % 4
            b1 = (3 + u) % 4
            b2 = u
            wait_scatter(b2)
            issue_load(t + 2, b2)
            wait_load(b1)
            issue_gather(b1)
            wait_gather(b0)
            compute(b0)
            issue_scatter(b0)
        return 0
    lax.fori_loop(0, (NT - 4) // 4, steady, 0)

    wait_scatter(0)
    wait_load(3)
    issue_gather(3)
    wait_gather(2)
    compute(2)
    issue_scatter(2)
    wait_gather(3)
    compute(3)
    issue_scatter(3)
    for b in range(1, 4):
        wait_scatter(b)

    plsc.subcore_barrier()

    def wcopy(j, _):
        base = s * 6256 + j * 368
        pltpu.sync_copy(acc.at[pl.ds(base, 368), :], zbuf)
        pltpu.sync_copy(zbuf, part_hbm.at[c, pl.ds(base, 368), :])
        return 0
    lax.fori_loop(0, 17, wcopy, 0)


ROWS_PER_W = N_PAD // NW
CB_CHUNK = 1564
CB_N = ROWS_PER_W // CB_CHUNK


@functools.partial(
    pl.kernel,
    out_type=(
        jax.ShapeDtypeStruct((N_PAD, C), _f32),
        jax.ShapeDtypeStruct((N_PAD, C), _f32),
    ),
    mesh=_mesh,
    compiler_params=_params,
    scratch_types=[
        pltpu.VMEM((16,), _f32),
        pltpu.VMEM((CB_CHUNK, C), _f32),
        pltpu.VMEM((CB_CHUNK, C), _f32),
        pltpu.VMEM((CB_CHUNK, C), _f32),
        pltpu.SemaphoreType.DMA,
    ],
)
def _combine_kernel(part_hbm, out_old_hbm, wk_hbm, x_hbm, out_hbm,
                    wkb, ab, bb, ob, lsem):
    c = lax.axis_index("c")
    s = lax.axis_index("s")
    wid = c * NS + s
    pltpu.sync_copy(wk_hbm, wkb)

    def chunk_body(ci, _):
        base = wid * ROWS_PER_W + ci * CB_CHUNK
        pltpu.async_copy(part_hbm.at[0, pl.ds(base, CB_CHUNK), :], ab, lsem)
        pltpu.async_copy(part_hbm.at[1, pl.ds(base, CB_CHUNK), :], bb, lsem)
        pltpu.async_copy(out_old_hbm.at[pl.ds(base, CB_CHUNK), :], ob, lsem)
        pltpu.make_async_copy(part_hbm.at[0, pl.ds(0, CB_CHUNK), :], ab,
                              lsem).wait()
        pltpu.make_async_copy(part_hbm.at[0, pl.ds(0, CB_CHUNK), :], bb,
                              lsem).wait()
        pltpu.make_async_copy(out_old_hbm.at[pl.ds(0, CB_CHUNK), :], ob,
                              lsem).wait()
        wk = wkb[...]

        def rbody(i, _):
            xv = ab[i, :] + bb[i, :]
            ab[i, :] = xv
            ob[i, :] = ob[i, :] + xv * wk
            return 0
        lax.fori_loop(0, CB_CHUNK, rbody, 0)
        pltpu.sync_copy(ab, x_hbm.at[pl.ds(base, CB_CHUNK), :])
        pltpu.sync_copy(ob, out_hbm.at[pl.ds(base, CB_CHUNK), :])
        return 0
    lax.fori_loop(0, CB_N, chunk_body, 0)


def kernel(edge_index, edge_attr, target, weight):
    row2d = edge_index[0].astype(_i32).reshape(NCH, CHUNK)
    col2d = edge_index[1].astype(_i32).reshape(NCH, CHUNK)
    attr2d = edge_attr.astype(_f32).reshape(NCH, CHUNK)
    degA, degB = _deg_kernel(row2d, attr2d)
    deg = _degsum_kernel(degA, degB)
    packi, packp = _pack_kernel(row2d, col2d, attr2d, deg)
    x = jnp.pad(target, ((0, N_PAD - N_NODES), (0, 0)))
    out = jnp.zeros((N_PAD, C), _f32)
    for k in range(weight.shape[1]):
        parts = _step_kernel(packi, packp, x)
        x, out = _combine_kernel(parts, out, weight[:, k])
    return out[:N_NODES]
